# baseline (speedup 1.0000x reference)
import functools

import jax
import jax.numpy as jnp
from jax.experimental import pallas as pl
from jax.experimental.pallas import tpu as pltpu

_B, _T, _C, _H, _W = 4, 16, 3, 224, 224
_DIM = 128
_HW = _H * _W
_G = _B * _T * _C
_BT = _B * _T
_GRID = 8
_RB = _G // _GRID

_KC = 8
_KF = 3


def _iota(shape, dim):
    return jax.lax.broadcasted_iota(jnp.int32, shape, dim)


def _topk_mask(s, k):
    t_io = _iota((_B, _T), 1)
    m = s
    mask = jnp.zeros((_B, _T), jnp.float32)
    idxs = []
    for _ in range(k):
        mx = jnp.max(m, axis=1, keepdims=True)
        cand = jnp.where(m == mx, t_io, _T)
        amin = jnp.min(cand, axis=1, keepdims=True)
        chosen = t_io == amin
        mask = jnp.where(chosen, 1.0, mask)
        m = jnp.where(chosen, -jnp.inf, m)
        idxs.append(amin)
    return mask, idxs


def _body(xr, w1, b1, wf, wd, bd, wc1, bc1, wc2, bc2, rnd,
          o_logits, o_s1, o_s2, o_fidx, acc):
    i = pl.program_id(0)
    acc[pl.ds(i * _RB, _RB), :] = (
        jnp.sum(xr[...], axis=2).sum(axis=1, keepdims=True))

    @pl.when(i == _GRID - 1)
    def _head():
        emb_col = acc[...] * (1.0 / _HW)

        r_io = _iota((_BT, _G), 1)
        bt_io = _iota((_BT, _G), 0)
        sel3 = (r_io // _C == bt_io).astype(jnp.float32)
        lc = (_iota((_G, _C), 0) % _C == _iota((_G, _C), 1)).astype(jnp.float32)
        embmat = jax.lax.dot(sel3, emb_col * lc,
                             preferred_element_type=jnp.float32)

        h = jax.nn.relu(jax.lax.dot(embmat, w1[...],
                                    preferred_element_type=jnp.float32) + b1[...])

        rep = (_iota((_BT, _B), 0) // _T == _iota((_BT, _B), 1)).astype(jnp.float32)
        sel4 = (_iota((_B, _BT), 1) // _T == _iota((_B, _BT), 0)).astype(jnp.float32)
        l16 = (_iota((_BT, _T), 0) % _T == _iota((_BT, _T), 1)).astype(jnp.float32)

        def scores(f):
            frep = jax.lax.dot(rep, f, preferred_element_type=jnp.float32)
            rs = jnp.sum(h * frep, axis=1, keepdims=True)
            return jax.lax.dot(sel4, rs * l16,
                               preferred_element_type=jnp.float32)

        def pooled_feats(mask, k):
            gm = jnp.concatenate([mask] * _B, axis=1) * sel4
            pooled = jax.lax.dot(gm, embmat,
                                 preferred_element_type=jnp.float32) * (1.0 / k)
            return jax.nn.relu(jax.lax.dot(pooled, wd[...],
                                           preferred_element_type=jnp.float32)
                               + bd[...])

        f1 = jax.lax.dot(rnd[...], wf[...], preferred_element_type=jnp.float32)
        s1 = scores(f1)
        mask1, _ = _topk_mask(s1, _KC)
        dama1 = pooled_feats(mask1, _KC)

        f2 = jax.lax.dot(dama1, wf[...], preferred_element_type=jnp.float32)
        s2 = scores(f2)
        mask2, idx2 = _topk_mask(s2, _KF)
        feats2 = pooled_feats(mask2, _KF)

        hc = jax.nn.relu(jax.lax.dot(feats2, wc1[...],
                                     preferred_element_type=jnp.float32) + bc1[...])
        logits = jax.lax.dot(hc, wc2[...],
                             preferred_element_type=jnp.float32) + bc2[...]

        o_logits[...] = logits
        o_s1[...] = s1
        o_s2[...] = s2
        o_fidx[...] = jnp.concatenate(idx2, axis=1)


@jax.jit
def kernel(x, W1_tks, b1_tks, Wf_tks, Wd, bd, Wc1, bc1, Wc2, bc2):
    xr = x.reshape(_G, _H, _W)
    rnd = jax.random.normal(jax.random.key(42), (_B, _DIM), dtype=jnp.float32)

    full = lambda shp: pl.BlockSpec(shp, lambda i: tuple(0 for _ in shp))
    out = pl.pallas_call(
        _body,
        grid=(_GRID,),
        in_specs=[
            pl.BlockSpec((_RB, _H, _W), lambda i: (i, 0, 0)),
            full((_C, _DIM)),
            full((1, _DIM)),
            full((_DIM, _DIM)),
            full((_C, _DIM)),
            full((1, _DIM)),
            full((_DIM, 256)),
            full((1, 256)),
            full((256, 2)),
            full((1, 2)),
            full((_B, _DIM)),
        ],
        out_specs=[
            full((_B, 2)),
            full((_B, _T)),
            full((_B, _T)),
            full((_B, _KF)),
        ],
        out_shape=[
            jax.ShapeDtypeStruct((_B, 2), jnp.float32),
            jax.ShapeDtypeStruct((_B, _T), jnp.float32),
            jax.ShapeDtypeStruct((_B, _T), jnp.float32),
            jax.ShapeDtypeStruct((_B, _KF), jnp.int32),
        ],
        scratch_shapes=[pltpu.VMEM((_G, 1), jnp.float32)],
    )(xr, W1_tks, b1_tks.reshape(1, _DIM), Wf_tks, Wd, bd.reshape(1, _DIM),
      Wc1, bc1.reshape(1, 256), Wc2, bc2.reshape(1, 2), rnd)
    logits, s1, s2, fidx = out
    return (logits, s1, s2, fidx)


# speedup vs baseline: 1.1368x; 1.1368x over previous
import functools

import jax
import jax.numpy as jnp
from jax.experimental import pallas as pl
from jax.experimental.pallas import tpu as pltpu

_B, _T, _C, _H, _W = 4, 16, 3, 224, 224
_DIM = 128
_HW = _H * _W
_G = _B * _T * _C
_BT = _B * _T
_GRID = 6
_RB = _G // _GRID

_KC = 8
_KF = 3


def _iota(shape, dim):
    return jax.lax.broadcasted_iota(jnp.int32, shape, dim)


def _dot(a, b):
    return jax.lax.dot(a, b, preferred_element_type=jnp.float32)


def _body(xr, w1, b1, wf, wd, bd, wc1, bc1, wc2, bc2, rnd,
          o_logits, o_s1, o_s2, o_fidx, acc):
    i = pl.program_id(0)
    acc[pl.ds(i * _RB, _RB), :] = (
        jnp.sum(xr[...], axis=2).sum(axis=1, keepdims=True))

    @pl.when(i == _GRID - 1)
    def _head():
        emb_col = acc[...] * (1.0 / _HW)

        r_io = _iota((_BT, _G), 1)
        bt_io = _iota((_BT, _G), 0)
        sel3 = (r_io // _C == bt_io).astype(jnp.float32)
        lc = (_iota((_G, _C), 0) % _C == _iota((_G, _C), 1)).astype(jnp.float32)
        embmat = _dot(sel3, emb_col * lc)

        h = jax.nn.relu(_dot(embmat, w1[...]) + b1[...])

        rep = (_iota((_BT, _B), 0) // _T == _iota((_BT, _B), 1)).astype(jnp.float32)
        sel4 = (_iota((_B, _BT), 1) // _T == _iota((_B, _BT), 0)).astype(jnp.float32)
        l16 = (_iota((_BT, _T), 0) % _T == _iota((_BT, _T), 1)).astype(jnp.float32)
        i_io = _iota((_BT, _T), 0) % _T
        j_io = _iota((_BT, _T), 1)

        def stage(f, k):
            frep = _dot(rep, f)
            rs = jnp.sum(h * frep, axis=1, keepdims=True)
            s2d = _dot(sel4, rs * l16)
            m1 = _dot(rep, s2d)
            cmp = (m1 > rs) | ((m1 == rs) & (j_io < i_io))
            rankcol = jnp.sum(cmp.astype(jnp.float32), axis=1, keepdims=True)
            maskcol = (rankcol < k).astype(jnp.float32)
            pooled = _dot(sel4, maskcol * embmat) * (1.0 / k)
            feats = jax.nn.relu(_dot(pooled, wd[...]) + bd[...])
            return s2d, rankcol, feats

        f1 = _dot(rnd[...], wf[...])
        s1, _, dama1 = stage(f1, _KC)

        f2 = _dot(dama1, wf[...])
        s2, rankcol2, feats2 = stage(f2, _KF)

        hc = jax.nn.relu(_dot(feats2, wc1[...]) + bc1[...])
        logits = _dot(hc, wc2[...]) + bc2[...]

        tcol = (_iota((_BT, _KF), 0) % _T).astype(jnp.float32)
        jio3 = _iota((_BT, _KF), 1).astype(jnp.float32)
        wm = jnp.where(rankcol2 == jio3, tcol, 0.0)
        fidx = _dot(sel4, wm)

        o_logits[...] = logits
        o_s1[...] = s1
        o_s2[...] = s2
        o_fidx[...] = fidx.astype(jnp.int32)


@jax.jit
def kernel(x, W1_tks, b1_tks, Wf_tks, Wd, bd, Wc1, bc1, Wc2, bc2):
    xr = x.reshape(_G, _H, _W)
    rnd = jax.random.normal(jax.random.key(42), (_B, _DIM), dtype=jnp.float32)

    full = lambda shp: pl.BlockSpec(shp, lambda i: tuple(0 for _ in shp))
    out = pl.pallas_call(
        _body,
        grid=(_GRID,),
        in_specs=[
            pl.BlockSpec((_RB, _H, _W), lambda i: (i, 0, 0)),
            full((_C, _DIM)),
            full((1, _DIM)),
            full((_DIM, _DIM)),
            full((_C, _DIM)),
            full((1, _DIM)),
            full((_DIM, 256)),
            full((1, 256)),
            full((256, 2)),
            full((1, 2)),
            full((_B, _DIM)),
        ],
        out_specs=[
            full((_B, 2)),
            full((_B, _T)),
            full((_B, _T)),
            full((_B, _KF)),
        ],
        out_shape=[
            jax.ShapeDtypeStruct((_B, 2), jnp.float32),
            jax.ShapeDtypeStruct((_B, _T), jnp.float32),
            jax.ShapeDtypeStruct((_B, _T), jnp.float32),
            jax.ShapeDtypeStruct((_B, _KF), jnp.int32),
        ],
        scratch_shapes=[pltpu.VMEM((_G, 1), jnp.float32)],
    )(xr, W1_tks, b1_tks.reshape(1, _DIM), Wf_tks, Wd, bd.reshape(1, _DIM),
      Wc1, bc1.reshape(1, 256), Wc2, bc2.reshape(1, 2), rnd)
    logits, s1, s2, fidx = out
    return (logits, s1, s2, fidx)
